# Initial kernel scaffold; baseline (speedup 1.0000x reference)
import jax
import jax.numpy as jnp
from jax.experimental import pallas as pl


def kernel(conv1_w, conv1_b, l0b0_w1, l0b0_b1, l0b0_w2, l0b0_b2, l0b0_w3, l0b0_b3, l0b0_wd, l0b0_bd, l0b1_w1, l0b1_b1, l0b1_w2, l0b1_b2, l0b1_w3, l0b1_b3, l0b2_w1, l0b2_b1, l0b2_w2, l0b2_b2, l0b2_w3, l0b2_b3, l1b0_w1, l1b0_b1, l1b0_w2, l1b0_b2, l1b0_w3, l1b0_b3, l1b0_wd, l1b0_bd, l1b1_w1, l1b1_b1, l1b1_w2, l1b1_b2, l1b1_w3, l1b1_b3, l1b2_w1, l1b2_b1, l1b2_w2, l1b2_b2, l1b2_w3, l1b2_b3, l1b3_w1, l1b3_b1, l1b3_w2, l1b3_b2, l1b3_w3, l1b3_b3, l2b0_w1, l2b0_b1, l2b0_w2, l2b0_b2, l2b0_w3, l2b0_b3, l2b0_wd, l2b0_bd, l2b1_w1, l2b1_b1, l2b1_w2, l2b1_b2, l2b1_w3, l2b1_b3, l2b2_w1, l2b2_b1, l2b2_w2, l2b2_b2, l2b2_w3, l2b2_b3, l2b3_w1, l2b3_b1, l2b3_w2, l2b3_b2, l2b3_w3, l2b3_b3, l2b4_w1, l2b4_b1, l2b4_w2, l2b4_b2, l2b4_w3, l2b4_b3, l2b5_w1, l2b5_b1, l2b5_w2, l2b5_b2, l2b5_w3, l2b5_b3, l2b6_w1, l2b6_b1, l2b6_w2, l2b6_b2, l2b6_w3, l2b6_b3, l2b7_w1, l2b7_b1, l2b7_w2, l2b7_b2, l2b7_w3, l2b7_b3, l2b8_w1, l2b8_b1, l2b8_w2, l2b8_b2, l2b8_w3, l2b8_b3, l2b9_w1, l2b9_b1, l2b9_w2, l2b9_b2, l2b9_w3, l2b9_b3, l2b10_w1, l2b10_b1, l2b10_w2, l2b10_b2, l2b10_w3, l2b10_b3, l2b11_w1, l2b11_b1, l2b11_w2, l2b11_b2, l2b11_w3, l2b11_b3, l2b12_w1, l2b12_b1, l2b12_w2, l2b12_b2, l2b12_w3, l2b12_b3, l2b13_w1, l2b13_b1, l2b13_w2, l2b13_b2, l2b13_w3, l2b13_b3, l2b14_w1, l2b14_b1, l2b14_w2, l2b14_b2, l2b14_w3, l2b14_b3, l2b15_w1, l2b15_b1, l2b15_w2, l2b15_b2, l2b15_w3, l2b15_b3, l2b16_w1, l2b16_b1, l2b16_w2, l2b16_b2, l2b16_w3, l2b16_b3, l2b17_w1, l2b17_b1, l2b17_w2, l2b17_b2, l2b17_w3, l2b17_b3, l2b18_w1, l2b18_b1, l2b18_w2, l2b18_b2, l2b18_w3, l2b18_b3, l2b19_w1, l2b19_b1, l2b19_w2, l2b19_b2, l2b19_w3, l2b19_b3, l2b20_w1, l2b20_b1, l2b20_w2, l2b20_b2, l2b20_w3, l2b20_b3, l2b21_w1, l2b21_b1, l2b21_w2, l2b21_b2, l2b21_w3, l2b21_b3, l2b22_w1, l2b22_b1, l2b22_w2, l2b22_b2, l2b22_w3, l2b22_b3, l3b0_w1, l3b0_b1, l3b0_w2, l3b0_b2, l3b0_w3, l3b0_b3, l3b0_wd, l3b0_bd, l3b1_w1, l3b1_b1, l3b1_w2, l3b1_b2, l3b1_w3, l3b1_b3, l3b2_w1, l3b2_b1, l3b2_w2, l3b2_b2, l3b2_w3, l3b2_b3, fc_w, fc_b, output, label):
    raise NotImplementedError("write your pallas kernel here")


# speedup vs baseline: 1.0237x; 1.0237x over previous
import functools

import jax
import jax.numpy as jnp
from jax.experimental import pallas as pl
from jax.experimental.pallas import tpu as pltpu

_BF = jnp.bfloat16
_F32 = jnp.float32


def _mm(x_bf, w, b, relu, residual=None):
    N = w.shape[-1]
    if N >= 512 and N % 256 == 0:
        parts = [jnp.dot(x_bf, w[:, j * 256:(j + 1) * 256],
                         preferred_element_type=_F32)
                 for j in range(N // 256)]
        acc = jnp.concatenate(parts, axis=-1)
    else:
        acc = jnp.dot(x_bf, w, preferred_element_type=_F32)
    acc = acc + b
    if residual is not None:
        acc = acc + residual
    if relu:
        acc = jnp.maximum(acc, 0.0)
    return acc


def _conv3x3_taps(h1, stride):
    B, H, W, P = h1.shape
    Ho = (H - 1) // stride + 1
    Wo = (W - 1) // stride + 1
    xp = jnp.pad(h1, ((0, 0), (1, 1), (1, 1), (0, 0)))
    taps = []
    for di in range(3):
        for dj in range(3):
            if stride == 1:
                tap = xp[:, di:di + H, dj:dj + W, :]
            else:
                tap = xp[:, di:di + 2 * Ho, dj:dj + 2 * Wo, :].reshape(
                    B, Ho, 2, Wo, 2, P)[:, :, 0, :, 0, :]
            taps.append(tap.reshape(B * Ho * Wo, P))
    return jnp.concatenate(taps, axis=-1)


def _mb_kernel(relu, has_res):
    def _kernel_body(*refs):
        if has_res:
            x_ref, w_ref, b_ref, r_ref, o_ref = refs
        else:
            x_ref, w_ref, b_ref, o_ref = refs
        acc = jnp.dot(x_ref[...], w_ref[...],
                      preferred_element_type=jnp.float32)
        acc = acc + b_ref[...]
        if has_res:
            acc = acc + r_ref[...]
        if relu:
            acc = jnp.maximum(acc, 0.0)
        o_ref[...] = acc
    return _kernel_body


def _pick_nt(N):
    if N >= 512 and N % 256 == 0:
        return 256
    return N


@functools.lru_cache(maxsize=None)
def _get_mb(M, K, N, relu, has_res):
    Nt = _pick_nt(N)
    in_specs = [
        pl.BlockSpec((M, K), lambda j: (0, 0)),
        pl.BlockSpec((K, Nt), lambda j: (0, j)),
        pl.BlockSpec((1, Nt), lambda j: (0, j)),
    ]
    if has_res:
        in_specs.append(pl.BlockSpec((M, Nt), lambda j: (0, j)))
    return pl.pallas_call(
        _mb_kernel(relu, has_res),
        out_shape=jax.ShapeDtypeStruct((M, N), jnp.float32),
        grid=(N // Nt,),
        in_specs=in_specs,
        out_specs=pl.BlockSpec((M, Nt), lambda j: (0, j)),
        compiler_params=pltpu.CompilerParams(
            dimension_semantics=("parallel",),
        ),
    )


def _mb(x, w, bias, relu, residual=None):
    M, K = x.shape
    N = w.shape[1]
    fn = _get_mb(int(M), int(K), int(N), bool(relu), residual is not None)
    args = [x.astype(_BF), w.astype(_BF), bias.reshape(1, N).astype(_F32)]
    if residual is not None:
        args.append(residual.reshape(M, N).astype(_F32))
    return fn(*args)


def _conv_mb(x, w, bias, kh, kw, stride, pad, relu, residual=None):
    B, H, W, C = x.shape
    Ho = (H + 2 * pad - kh) // stride + 1
    Wo = (W + 2 * pad - kw) // stride + 1
    xp = jnp.pad(x, ((0, 0), (pad, pad), (pad, pad), (0, 0))) if pad > 0 else x
    cols = []
    for di in range(kh):
        for dj in range(kw):
            cols.append(xp[:, di:di + stride * Ho:stride,
                            dj:dj + stride * Wo:stride, :])
    patches = jnp.concatenate(cols, axis=-1) if len(cols) > 1 else cols[0]
    pm = patches.reshape(B * Ho * Wo, kh * kw * C)
    res = None
    if residual is not None:
        res = residual.reshape(B * Ho * Wo, w.shape[1])
    out = _mb(pm, w, bias, relu, res)
    return out.reshape(B, Ho, Wo, w.shape[1])


def _stem_body(p_ref, w_ref, b_ref, o_ref):
    TB = o_ref.shape[0]
    conv = _mm(p_ref[...], w_ref[...], b_ref[...], relu=True)
    conv = conv.reshape(TB, 32, 32, 64)
    xp = jnp.pad(conv, ((0, 0), (1, 1), (1, 1), (0, 0)))
    m = None
    for di in range(3):
        for dj in range(3):
            tap = xp[:, di:di + 32, dj:dj + 32, :].reshape(
                TB, 16, 2, 16, 2, 64)[:, :, 0, :, 0, :]
            m = tap if m is None else jnp.maximum(m, tap)
    o_ref[...] = m


def _stem(patches, w, b):
    TB = patches.shape[0] // 1024
    return pl.pallas_call(
        _stem_body,
        in_specs=[pl.BlockSpec(memory_space=pltpu.MemorySpace.VMEM)] * 3,
        out_specs=pl.BlockSpec(memory_space=pltpu.MemorySpace.VMEM),
        out_shape=jax.ShapeDtypeStruct((TB, 16, 16, 64), _F32),
    )(patches, w, b)


def _stem_patches(x):
    TB = x.shape[0]
    xp = jnp.pad(x, ((0, 0), (3, 3), (3, 3), (0, 0)))
    cols = []
    for di in range(7):
        for dj in range(7):
            cols.append(xp[:, di:di + 64:2, dj:dj + 64:2, :])
    return jnp.concatenate(cols, axis=-1).astype(_BF).reshape(TB * 1024, 98)


def _block0_body(stride):
    def body(x_ref, w1_ref, b1_ref, w2_ref, b2_ref, w3_ref, b3_ref,
             r_ref, o_ref):
        x4 = x_ref[...]
        B, H, W, C = x4.shape
        h1 = _mm(x4.reshape(B * H * W, C).astype(_BF), w1_ref[...],
                 b1_ref[...], relu=True)
        h1 = h1.astype(_BF).reshape(B, H, W, -1)
        h2 = _mm(_conv3x3_taps(h1, stride), w2_ref[...], b2_ref[...],
                 relu=True)
        Mo = h2.shape[0]
        h3 = _mm(h2.astype(_BF), w3_ref[...], b3_ref[...], relu=True,
                 residual=r_ref[...].reshape(Mo, -1))
        o_ref[...] = h3.reshape(o_ref.shape)
    return body


def _block0(x, w1, b1, w2, b2, w3, b3, wd, bd, stride):
    TB, H, W, C = x.shape
    P = w1.shape[-1]
    N3 = w3.shape[-1]
    Ho = (H - 1) // stride + 1
    Wo = (W - 1) // stride + 1
    if stride == 2:
        xd = x[:, ::2, ::2, :]
    else:
        xd = x
    idn = _mb(xd.reshape(TB * Ho * Wo, C), wd, bd, relu=False)
    return pl.pallas_call(
        _block0_body(stride),
        in_specs=[pl.BlockSpec(memory_space=pltpu.MemorySpace.VMEM)] * 8,
        out_specs=pl.BlockSpec(memory_space=pltpu.MemorySpace.VMEM),
        out_shape=jax.ShapeDtypeStruct((TB, Ho, Wo, N3), _F32),
    )(x, w1, b1.reshape(1, P), w2, b2.reshape(1, P), w3, b3.reshape(1, N3),
      idn.reshape(TB, Ho, Wo, N3))


def _rest_body(nb1):
    def body(x_ref, w1_ref, b1_ref, w2_ref, b2_ref, w3_ref, b3_ref,
             o_ref, xs_ref):
        b = pl.program_id(0)

        @pl.when(b == 0)
        def _():
            xs_ref[...] = x_ref[...]

        x4 = xs_ref[...]
        B, H, W, C = x4.shape
        M = B * H * W
        h1 = _mm(x4.reshape(M, C).astype(_BF), w1_ref[0], b1_ref[0],
                 relu=True)
        h1 = h1.astype(_BF).reshape(B, H, W, -1)
        h2 = _mm(_conv3x3_taps(h1, 1), w2_ref[0], b2_ref[0], relu=True)
        h3 = _mm(h2.astype(_BF), w3_ref[0], b3_ref[0], relu=True,
                 residual=x4.reshape(M, C))
        out = h3.reshape(B, H, W, C)
        xs_ref[...] = out

        @pl.when(b == nb1 - 1)
        def _():
            o_ref[...] = out
    return body


def _stage_rest(x, w1s, b1s, w2s, b2s, w3s, b3s):
    TB, H, W, C = x.shape
    nb1, _, P = w1s.shape
    return pl.pallas_call(
        _rest_body(nb1),
        grid=(nb1,),
        in_specs=[
            pl.BlockSpec((TB, H, W, C), lambda b: (0, 0, 0, 0)),
            pl.BlockSpec((1, C, P), lambda b: (b, 0, 0)),
            pl.BlockSpec((1, 1, P), lambda b: (b, 0, 0)),
            pl.BlockSpec((1, 9 * P, P), lambda b: (b, 0, 0)),
            pl.BlockSpec((1, 1, P), lambda b: (b, 0, 0)),
            pl.BlockSpec((1, P, C), lambda b: (b, 0, 0)),
            pl.BlockSpec((1, 1, C), lambda b: (b, 0, 0)),
        ],
        out_specs=pl.BlockSpec((TB, H, W, C), lambda b: (0, 0, 0, 0)),
        out_shape=jax.ShapeDtypeStruct((TB, H, W, C), _F32),
        scratch_shapes=[pltpu.VMEM((TB, H, W, C), _F32)],
        compiler_params=pltpu.CompilerParams(
            dimension_semantics=("arbitrary",)),
    )(x, w1s, b1s, w2s, b2s, w3s, b3s)


@functools.lru_cache(maxsize=None)
def _get_fc_mse(M, K, C, nb):
    inv_n = 1.0 / float(nb * C)

    def _kernel_body(x_ref, w_ref, b_ref, o_ref):
        logits = jnp.dot(x_ref[...], w_ref[...],
                         preferred_element_type=jnp.float32)
        logits = logits + b_ref[...]
        d = logits[:nb] - logits[nb:]
        o_ref[0] = jnp.sum(d * d) * inv_n

    return pl.pallas_call(
        _kernel_body,
        out_shape=jax.ShapeDtypeStruct((1,), jnp.float32),
        in_specs=[pl.BlockSpec(memory_space=pltpu.MemorySpace.VMEM),
                  pl.BlockSpec(memory_space=pltpu.MemorySpace.VMEM),
                  pl.BlockSpec(memory_space=pltpu.MemorySpace.VMEM)],
        out_specs=pl.BlockSpec(memory_space=pltpu.MemorySpace.SMEM),
    )


_LAYER_CFG = ((3, 64), (4, 128), (23, 256), (3, 512))


def _bottleneck_ref(x, bp, stride):
    out = _conv_mb(x, bp['w1'], bp['b1'], 1, 1, 1, 0, relu=True)
    out = _conv_mb(out, bp['w2'], bp['b2'], 3, 3, stride, 1, relu=True)
    if 'wd' in bp:
        identity = _conv_mb(x, bp['wd'], bp['bd'], 1, 1, stride, 0,
                            relu=False)
    else:
        identity = x
    return _conv_mb(out, bp['w3'], bp['b3'], 1, 1, 1, 0, relu=True,
                    residual=identity)


def kernel(conv1_w, conv1_b, l0b0_w1, l0b0_b1, l0b0_w2, l0b0_b2, l0b0_w3, l0b0_b3, l0b0_wd, l0b0_bd, l0b1_w1, l0b1_b1, l0b1_w2, l0b1_b2, l0b1_w3, l0b1_b3, l0b2_w1, l0b2_b1, l0b2_w2, l0b2_b2, l0b2_w3, l0b2_b3, l1b0_w1, l1b0_b1, l1b0_w2, l1b0_b2, l1b0_w3, l1b0_b3, l1b0_wd, l1b0_bd, l1b1_w1, l1b1_b1, l1b1_w2, l1b1_b2, l1b1_w3, l1b1_b3, l1b2_w1, l1b2_b1, l1b2_w2, l1b2_b2, l1b2_w3, l1b2_b3, l1b3_w1, l1b3_b1, l1b3_w2, l1b3_b2, l1b3_w3, l1b3_b3, l2b0_w1, l2b0_b1, l2b0_w2, l2b0_b2, l2b0_w3, l2b0_b3, l2b0_wd, l2b0_bd, l2b1_w1, l2b1_b1, l2b1_w2, l2b1_b2, l2b1_w3, l2b1_b3, l2b2_w1, l2b2_b1, l2b2_w2, l2b2_b2, l2b2_w3, l2b2_b3, l2b3_w1, l2b3_b1, l2b3_w2, l2b3_b2, l2b3_w3, l2b3_b3, l2b4_w1, l2b4_b1, l2b4_w2, l2b4_b2, l2b4_w3, l2b4_b3, l2b5_w1, l2b5_b1, l2b5_w2, l2b5_b2, l2b5_w3, l2b5_b3, l2b6_w1, l2b6_b1, l2b6_w2, l2b6_b2, l2b6_w3, l2b6_b3, l2b7_w1, l2b7_b1, l2b7_w2, l2b7_b2, l2b7_w3, l2b7_b3, l2b8_w1, l2b8_b1, l2b8_w2, l2b8_b2, l2b8_w3, l2b8_b3, l2b9_w1, l2b9_b1, l2b9_w2, l2b9_b2, l2b9_w3, l2b9_b3, l2b10_w1, l2b10_b1, l2b10_w2, l2b10_b2, l2b10_w3, l2b10_b3, l2b11_w1, l2b11_b1, l2b11_w2, l2b11_b2, l2b11_w3, l2b11_b3, l2b12_w1, l2b12_b1, l2b12_w2, l2b12_b2, l2b12_w3, l2b12_b3, l2b13_w1, l2b13_b1, l2b13_w2, l2b13_b2, l2b13_w3, l2b13_b3, l2b14_w1, l2b14_b1, l2b14_w2, l2b14_b2, l2b14_w3, l2b14_b3, l2b15_w1, l2b15_b1, l2b15_w2, l2b15_b2, l2b15_w3, l2b15_b3, l2b16_w1, l2b16_b1, l2b16_w2, l2b16_b2, l2b16_w3, l2b16_b3, l2b17_w1, l2b17_b1, l2b17_w2, l2b17_b2, l2b17_w3, l2b17_b3, l2b18_w1, l2b18_b1, l2b18_w2, l2b18_b2, l2b18_w3, l2b18_b3, l2b19_w1, l2b19_b1, l2b19_w2, l2b19_b2, l2b19_w3, l2b19_b3, l2b20_w1, l2b20_b1, l2b20_w2, l2b20_b2, l2b20_w3, l2b20_b3, l2b21_w1, l2b21_b1, l2b21_w2, l2b21_b2, l2b21_w3, l2b21_b3, l2b22_w1, l2b22_b1, l2b22_w2, l2b22_b2, l2b22_w3, l2b22_b3, l3b0_w1, l3b0_b1, l3b0_w2, l3b0_b2, l3b0_w3, l3b0_b3, l3b0_wd, l3b0_bd, l3b1_w1, l3b1_b1, l3b1_w2, l3b1_b2, l3b1_w3, l3b1_b3, l3b2_w1, l3b2_b1, l3b2_w2, l3b2_b2, l3b2_w3, l3b2_b3, fc_w, fc_b, output, label):
    _L = locals()
    nb = int(output.shape[0])

    x = jnp.concatenate([output, label], axis=0).astype(_F32)
    x = jnp.transpose(x, (0, 2, 3, 1))

    x = _stem(_stem_patches(x), conv1_w, conv1_b.reshape(1, 64))

    for li, (nblocks, planes) in enumerate(_LAYER_CFG):
        stride = 1 if li == 0 else 2
        for bi in range(nblocks):
            bp = {'w1': _L[f"l{li}b{bi}_w1"], 'b1': _L[f"l{li}b{bi}_b1"],
                  'w2': _L[f"l{li}b{bi}_w2"], 'b2': _L[f"l{li}b{bi}_b2"],
                  'w3': _L[f"l{li}b{bi}_w3"], 'b3': _L[f"l{li}b{bi}_b3"]}
            if f"l{li}b{bi}_wd" in _L and bi == 0:
                bp['wd'] = _L[f"l{li}b0_wd"]
                bp['bd'] = _L[f"l{li}b0_bd"]
            x = _bottleneck_ref(x, bp, stride if bi == 0 else 1)

    feats = jnp.mean(x, axis=(1, 2))
    M, K = int(feats.shape[0]), int(feats.shape[1])
    C = int(fc_w.shape[1])
    fn = _get_fc_mse(M, K, C, nb)
    return fn(feats.astype(_BF), fc_w.astype(_BF),
              fc_b.reshape(1, C).astype(_F32))[0]


# speedup vs baseline: 1.0877x; 1.0626x over previous
import functools

import jax
import jax.numpy as jnp
from jax.experimental import pallas as pl
from jax.experimental.pallas import tpu as pltpu

_BF = jnp.bfloat16
_F32 = jnp.float32


def _mm(x_bf, w, b, relu, residual=None):
    N = w.shape[-1]
    if N >= 512 and N % 256 == 0:
        parts = [jnp.dot(x_bf, w[:, j * 256:(j + 1) * 256],
                         preferred_element_type=_F32)
                 for j in range(N // 256)]
        acc = jnp.concatenate(parts, axis=-1)
    else:
        acc = jnp.dot(x_bf, w, preferred_element_type=_F32)
    acc = acc + b
    if residual is not None:
        acc = acc + residual
    if relu:
        acc = jnp.maximum(acc, 0.0)
    return acc


def _conv3x3_taps(h1, stride):
    B, H, W, P = h1.shape
    Ho = (H - 1) // stride + 1
    Wo = (W - 1) // stride + 1
    xp = jnp.pad(h1, ((0, 0), (1, 1), (1, 1), (0, 0)))
    taps = []
    for di in range(3):
        for dj in range(3):
            if stride == 1:
                tap = xp[:, di:di + H, dj:dj + W, :]
            else:
                tap = xp[:, di:di + 2 * Ho, dj:dj + 2 * Wo, :].reshape(
                    B, Ho, 2, Wo, 2, P)[:, :, 0, :, 0, :]
            taps.append(tap.reshape(B * Ho * Wo, P))
    return jnp.concatenate(taps, axis=-1)


def _mb_kernel(relu, has_res):
    def _kernel_body(*refs):
        if has_res:
            x_ref, w_ref, b_ref, r_ref, o_ref = refs
        else:
            x_ref, w_ref, b_ref, o_ref = refs
        acc = jnp.dot(x_ref[...], w_ref[...],
                      preferred_element_type=jnp.float32)
        acc = acc + b_ref[...]
        if has_res:
            acc = acc + r_ref[...]
        if relu:
            acc = jnp.maximum(acc, 0.0)
        o_ref[...] = acc
    return _kernel_body


def _pick_nt(N):
    if N >= 512 and N % 256 == 0:
        return 256
    return N


@functools.lru_cache(maxsize=None)
def _get_mb(M, K, N, relu, has_res):
    Nt = _pick_nt(N)
    in_specs = [
        pl.BlockSpec((M, K), lambda j: (0, 0)),
        pl.BlockSpec((K, Nt), lambda j: (0, j)),
        pl.BlockSpec((1, Nt), lambda j: (0, j)),
    ]
    if has_res:
        in_specs.append(pl.BlockSpec((M, Nt), lambda j: (0, j)))
    return pl.pallas_call(
        _mb_kernel(relu, has_res),
        out_shape=jax.ShapeDtypeStruct((M, N), jnp.float32),
        grid=(N // Nt,),
        in_specs=in_specs,
        out_specs=pl.BlockSpec((M, Nt), lambda j: (0, j)),
        compiler_params=pltpu.CompilerParams(
            dimension_semantics=("parallel",),
        ),
    )


def _mb(x, w, bias, relu, residual=None):
    M, K = x.shape
    N = w.shape[1]
    fn = _get_mb(int(M), int(K), int(N), bool(relu), residual is not None)
    args = [x.astype(_BF), w.astype(_BF), bias.reshape(1, N).astype(_F32)]
    if residual is not None:
        args.append(residual.reshape(M, N).astype(_F32))
    return fn(*args)


def _conv_mb(x, w, bias, kh, kw, stride, pad, relu, residual=None):
    B, H, W, C = x.shape
    Ho = (H + 2 * pad - kh) // stride + 1
    Wo = (W + 2 * pad - kw) // stride + 1
    xp = jnp.pad(x, ((0, 0), (pad, pad), (pad, pad), (0, 0))) if pad > 0 else x
    cols = []
    for di in range(kh):
        for dj in range(kw):
            cols.append(xp[:, di:di + stride * Ho:stride,
                            dj:dj + stride * Wo:stride, :])
    patches = jnp.concatenate(cols, axis=-1) if len(cols) > 1 else cols[0]
    pm = patches.reshape(B * Ho * Wo, kh * kw * C)
    res = None
    if residual is not None:
        res = residual.reshape(B * Ho * Wo, w.shape[1])
    out = _mb(pm, w, bias, relu, res)
    return out.reshape(B, Ho, Wo, w.shape[1])


def _stem_body(p_ref, w_ref, b_ref, o_ref):
    TB = o_ref.shape[0]
    conv = _mm(p_ref[...], w_ref[...], b_ref[...], relu=True)
    conv = conv.reshape(TB, 32, 32, 64)
    xp = jnp.pad(conv, ((0, 0), (1, 1), (1, 1), (0, 0)))
    m = None
    for di in range(3):
        for dj in range(3):
            tap = xp[:, di:di + 32, dj:dj + 32, :].reshape(
                TB, 16, 2, 16, 2, 64)[:, :, 0, :, 0, :]
            m = tap if m is None else jnp.maximum(m, tap)
    o_ref[...] = m


def _stem(patches, w, b):
    TB = patches.shape[0] // 1024
    return pl.pallas_call(
        _stem_body,
        in_specs=[pl.BlockSpec(memory_space=pltpu.MemorySpace.VMEM)] * 3,
        out_specs=pl.BlockSpec(memory_space=pltpu.MemorySpace.VMEM),
        out_shape=jax.ShapeDtypeStruct((TB, 16, 16, 64), _F32),
    )(patches, w, b)


def _stem_patches(x):
    TB = x.shape[0]
    xp = jnp.pad(x, ((0, 0), (3, 3), (3, 3), (0, 0)))
    cols = []
    for di in range(7):
        for dj in range(7):
            cols.append(xp[:, di:di + 64:2, dj:dj + 64:2, :])
    return jnp.concatenate(cols, axis=-1).astype(_BF).reshape(TB * 1024, 98)


def _block0_body(stride):
    def body(x_ref, w1_ref, b1_ref, w2_ref, b2_ref, w3_ref, b3_ref,
             r_ref, o_ref):
        x4 = x_ref[...]
        B, H, W, C = x4.shape
        h1 = _mm(x4.reshape(B * H * W, C).astype(_BF), w1_ref[...],
                 b1_ref[...], relu=True)
        h1 = h1.astype(_BF).reshape(B, H, W, -1)
        h2 = _mm(_conv3x3_taps(h1, stride), w2_ref[...], b2_ref[...],
                 relu=True)
        Mo = h2.shape[0]
        h3 = _mm(h2.astype(_BF), w3_ref[...], b3_ref[...], relu=True,
                 residual=r_ref[...].reshape(Mo, -1))
        o_ref[...] = h3.reshape(o_ref.shape)
    return body


def _block0(x, w1, b1, w2, b2, w3, b3, wd, bd, stride):
    TB, H, W, C = x.shape
    P = w1.shape[-1]
    N3 = w3.shape[-1]
    Ho = (H - 1) // stride + 1
    Wo = (W - 1) // stride + 1
    if stride == 2:
        xd = x[:, ::2, ::2, :]
    else:
        xd = x
    idn = _mb(xd.reshape(TB * Ho * Wo, C), wd, bd, relu=False)
    return pl.pallas_call(
        _block0_body(stride),
        in_specs=[pl.BlockSpec(memory_space=pltpu.MemorySpace.VMEM)] * 8,
        out_specs=pl.BlockSpec(memory_space=pltpu.MemorySpace.VMEM),
        out_shape=jax.ShapeDtypeStruct((TB, Ho, Wo, N3), _F32),
    )(x, w1, b1.reshape(1, P), w2, b2.reshape(1, P), w3, b3.reshape(1, N3),
      idn.reshape(TB, Ho, Wo, N3))


def _rest_body(nb1):
    def body(x_ref, w1_ref, b1_ref, w2_ref, b2_ref, w3_ref, b3_ref,
             o_ref, xs_ref):
        b = pl.program_id(0)

        @pl.when(b == 0)
        def _():
            xs_ref[...] = x_ref[...]

        x4 = xs_ref[...]
        B, H, W, C = x4.shape
        M = B * H * W
        h1 = _mm(x4.reshape(M, C).astype(_BF), w1_ref[0], b1_ref[0],
                 relu=True)
        h1 = h1.astype(_BF).reshape(B, H, W, -1)
        h2 = _mm(_conv3x3_taps(h1, 1), w2_ref[0], b2_ref[0], relu=True)
        h3 = _mm(h2.astype(_BF), w3_ref[0], b3_ref[0], relu=True,
                 residual=x4.reshape(M, C))
        out = h3.reshape(B, H, W, C)
        xs_ref[...] = out

        @pl.when(b == nb1 - 1)
        def _():
            o_ref[...] = out
    return body


def _stage_rest(x, w1s, b1s, w2s, b2s, w3s, b3s):
    TB, H, W, C = x.shape
    nb1, _, P = w1s.shape
    return pl.pallas_call(
        _rest_body(nb1),
        grid=(nb1,),
        in_specs=[
            pl.BlockSpec((TB, H, W, C), lambda b: (0, 0, 0, 0)),
            pl.BlockSpec((1, C, P), lambda b: (b, 0, 0)),
            pl.BlockSpec((1, 1, P), lambda b: (b, 0, 0)),
            pl.BlockSpec((1, 9 * P, P), lambda b: (b, 0, 0)),
            pl.BlockSpec((1, 1, P), lambda b: (b, 0, 0)),
            pl.BlockSpec((1, P, C), lambda b: (b, 0, 0)),
            pl.BlockSpec((1, 1, C), lambda b: (b, 0, 0)),
        ],
        out_specs=pl.BlockSpec((TB, H, W, C), lambda b: (0, 0, 0, 0)),
        out_shape=jax.ShapeDtypeStruct((TB, H, W, C), _F32),
        scratch_shapes=[pltpu.VMEM((TB, H, W, C), _F32)],
        compiler_params=pltpu.CompilerParams(
            dimension_semantics=("arbitrary",)),
    )(x, w1s, b1s, w2s, b2s, w3s, b3s)


@functools.lru_cache(maxsize=None)
def _get_fc_mse(M, K, C, nb):
    inv_n = 1.0 / float(nb * C)

    def _kernel_body(x_ref, w_ref, b_ref, o_ref):
        logits = jnp.dot(x_ref[...], w_ref[...],
                         preferred_element_type=jnp.float32)
        logits = logits + b_ref[...]
        d = logits[:nb] - logits[nb:]
        o_ref[0] = jnp.sum(d * d) * inv_n

    return pl.pallas_call(
        _kernel_body,
        out_shape=jax.ShapeDtypeStruct((1,), jnp.float32),
        in_specs=[pl.BlockSpec(memory_space=pltpu.MemorySpace.VMEM),
                  pl.BlockSpec(memory_space=pltpu.MemorySpace.VMEM),
                  pl.BlockSpec(memory_space=pltpu.MemorySpace.VMEM)],
        out_specs=pl.BlockSpec(memory_space=pltpu.MemorySpace.SMEM),
    )


_LAYER_CFG = ((3, 64), (4, 128), (23, 256), (3, 512))


def _bottleneck_ref(x, bp, stride):
    out = _conv_mb(x, bp['w1'], bp['b1'], 1, 1, 1, 0, relu=True)
    out = _conv_mb(out, bp['w2'], bp['b2'], 3, 3, stride, 1, relu=True)
    if 'wd' in bp:
        identity = _conv_mb(x, bp['wd'], bp['bd'], 1, 1, stride, 0,
                            relu=False)
    else:
        identity = x
    return _conv_mb(out, bp['w3'], bp['b3'], 1, 1, 1, 0, relu=True,
                    residual=identity)


def kernel(conv1_w, conv1_b, l0b0_w1, l0b0_b1, l0b0_w2, l0b0_b2, l0b0_w3, l0b0_b3, l0b0_wd, l0b0_bd, l0b1_w1, l0b1_b1, l0b1_w2, l0b1_b2, l0b1_w3, l0b1_b3, l0b2_w1, l0b2_b1, l0b2_w2, l0b2_b2, l0b2_w3, l0b2_b3, l1b0_w1, l1b0_b1, l1b0_w2, l1b0_b2, l1b0_w3, l1b0_b3, l1b0_wd, l1b0_bd, l1b1_w1, l1b1_b1, l1b1_w2, l1b1_b2, l1b1_w3, l1b1_b3, l1b2_w1, l1b2_b1, l1b2_w2, l1b2_b2, l1b2_w3, l1b2_b3, l1b3_w1, l1b3_b1, l1b3_w2, l1b3_b2, l1b3_w3, l1b3_b3, l2b0_w1, l2b0_b1, l2b0_w2, l2b0_b2, l2b0_w3, l2b0_b3, l2b0_wd, l2b0_bd, l2b1_w1, l2b1_b1, l2b1_w2, l2b1_b2, l2b1_w3, l2b1_b3, l2b2_w1, l2b2_b1, l2b2_w2, l2b2_b2, l2b2_w3, l2b2_b3, l2b3_w1, l2b3_b1, l2b3_w2, l2b3_b2, l2b3_w3, l2b3_b3, l2b4_w1, l2b4_b1, l2b4_w2, l2b4_b2, l2b4_w3, l2b4_b3, l2b5_w1, l2b5_b1, l2b5_w2, l2b5_b2, l2b5_w3, l2b5_b3, l2b6_w1, l2b6_b1, l2b6_w2, l2b6_b2, l2b6_w3, l2b6_b3, l2b7_w1, l2b7_b1, l2b7_w2, l2b7_b2, l2b7_w3, l2b7_b3, l2b8_w1, l2b8_b1, l2b8_w2, l2b8_b2, l2b8_w3, l2b8_b3, l2b9_w1, l2b9_b1, l2b9_w2, l2b9_b2, l2b9_w3, l2b9_b3, l2b10_w1, l2b10_b1, l2b10_w2, l2b10_b2, l2b10_w3, l2b10_b3, l2b11_w1, l2b11_b1, l2b11_w2, l2b11_b2, l2b11_w3, l2b11_b3, l2b12_w1, l2b12_b1, l2b12_w2, l2b12_b2, l2b12_w3, l2b12_b3, l2b13_w1, l2b13_b1, l2b13_w2, l2b13_b2, l2b13_w3, l2b13_b3, l2b14_w1, l2b14_b1, l2b14_w2, l2b14_b2, l2b14_w3, l2b14_b3, l2b15_w1, l2b15_b1, l2b15_w2, l2b15_b2, l2b15_w3, l2b15_b3, l2b16_w1, l2b16_b1, l2b16_w2, l2b16_b2, l2b16_w3, l2b16_b3, l2b17_w1, l2b17_b1, l2b17_w2, l2b17_b2, l2b17_w3, l2b17_b3, l2b18_w1, l2b18_b1, l2b18_w2, l2b18_b2, l2b18_w3, l2b18_b3, l2b19_w1, l2b19_b1, l2b19_w2, l2b19_b2, l2b19_w3, l2b19_b3, l2b20_w1, l2b20_b1, l2b20_w2, l2b20_b2, l2b20_w3, l2b20_b3, l2b21_w1, l2b21_b1, l2b21_w2, l2b21_b2, l2b21_w3, l2b21_b3, l2b22_w1, l2b22_b1, l2b22_w2, l2b22_b2, l2b22_w3, l2b22_b3, l3b0_w1, l3b0_b1, l3b0_w2, l3b0_b2, l3b0_w3, l3b0_b3, l3b0_wd, l3b0_bd, l3b1_w1, l3b1_b1, l3b1_w2, l3b1_b2, l3b1_w3, l3b1_b3, l3b2_w1, l3b2_b1, l3b2_w2, l3b2_b2, l3b2_w3, l3b2_b3, fc_w, fc_b, output, label):
    _L = locals()
    nb = int(output.shape[0])

    x = jnp.concatenate([output, label], axis=0).astype(_F32)
    x = jnp.transpose(x, (0, 2, 3, 1))

    x = _stem(_stem_patches(x), conv1_w, conv1_b.reshape(1, 64))

    for li, (nblocks, planes) in enumerate(_LAYER_CFG):
        stride = 1 if li == 0 else 2
        if li == 0:
            P, C = planes, planes * 4
            x = _block0(x,
                        _L["l0b0_w1"], _L["l0b0_b1"], _L["l0b0_w2"],
                        _L["l0b0_b2"], _L["l0b0_w3"], _L["l0b0_b3"],
                        _L["l0b0_wd"], _L["l0b0_bd"], stride)
            for b in range(1, nblocks):
                x = _stage_rest(x,
                                _L[f"l0b{b}_w1"].reshape(1, C, P),
                                _L[f"l0b{b}_b1"].reshape(1, 1, P),
                                _L[f"l0b{b}_w2"].reshape(1, 9 * P, P),
                                _L[f"l0b{b}_b2"].reshape(1, 1, P),
                                _L[f"l0b{b}_w3"].reshape(1, P, C),
                                _L[f"l0b{b}_b3"].reshape(1, 1, C))
            continue
        for bi in range(nblocks):
            bp = {'w1': _L[f"l{li}b{bi}_w1"], 'b1': _L[f"l{li}b{bi}_b1"],
                  'w2': _L[f"l{li}b{bi}_w2"], 'b2': _L[f"l{li}b{bi}_b2"],
                  'w3': _L[f"l{li}b{bi}_w3"], 'b3': _L[f"l{li}b{bi}_b3"]}
            if f"l{li}b{bi}_wd" in _L and bi == 0:
                bp['wd'] = _L[f"l{li}b0_wd"]
                bp['bd'] = _L[f"l{li}b0_bd"]
            x = _bottleneck_ref(x, bp, stride if bi == 0 else 1)

    feats = jnp.mean(x, axis=(1, 2))
    M, K = int(feats.shape[0]), int(feats.shape[1])
    C = int(fc_w.shape[1])
    fn = _get_fc_mse(M, K, C, nb)
    return fn(feats.astype(_BF), fc_w.astype(_BF),
              fc_b.reshape(1, C).astype(_F32))[0]


# speedup vs baseline: 1.2923x; 1.1881x over previous
import functools

import jax
import jax.numpy as jnp
from jax.experimental import pallas as pl
from jax.experimental.pallas import tpu as pltpu

_BF = jnp.bfloat16
_F32 = jnp.float32


def _mm(x_bf, w, b, relu, residual=None):
    N = w.shape[-1]
    if N >= 512 and N % 256 == 0:
        parts = [jnp.dot(x_bf, w[:, j * 256:(j + 1) * 256],
                         preferred_element_type=_F32)
                 for j in range(N // 256)]
        acc = jnp.concatenate(parts, axis=-1)
    else:
        acc = jnp.dot(x_bf, w, preferred_element_type=_F32)
    acc = acc + b
    if residual is not None:
        acc = acc + residual
    if relu:
        acc = jnp.maximum(acc, 0.0)
    return acc


def _conv3x3_taps(h1, stride):
    B, H, W, P = h1.shape
    Ho = (H - 1) // stride + 1
    Wo = (W - 1) // stride + 1
    xp = jnp.pad(h1, ((0, 0), (1, 1), (1, 1), (0, 0)))
    taps = []
    for di in range(3):
        for dj in range(3):
            if stride == 1:
                tap = xp[:, di:di + H, dj:dj + W, :]
            else:
                tap = xp[:, di:di + 2 * Ho, dj:dj + 2 * Wo, :].reshape(
                    B, Ho, 2, Wo, 2, P)[:, :, 0, :, 0, :]
            taps.append(tap.reshape(B * Ho * Wo, P))
    return jnp.concatenate(taps, axis=-1)


def _mb_kernel(relu, has_res):
    def _kernel_body(*refs):
        if has_res:
            x_ref, w_ref, b_ref, r_ref, o_ref = refs
        else:
            x_ref, w_ref, b_ref, o_ref = refs
        acc = jnp.dot(x_ref[...], w_ref[...],
                      preferred_element_type=jnp.float32)
        acc = acc + b_ref[...]
        if has_res:
            acc = acc + r_ref[...]
        if relu:
            acc = jnp.maximum(acc, 0.0)
        o_ref[...] = acc
    return _kernel_body


def _pick_nt(N):
    if N >= 512 and N % 256 == 0:
        return 256
    return N


@functools.lru_cache(maxsize=None)
def _get_mb(M, K, N, relu, has_res):
    Nt = _pick_nt(N)
    in_specs = [
        pl.BlockSpec((M, K), lambda j: (0, 0)),
        pl.BlockSpec((K, Nt), lambda j: (0, j)),
        pl.BlockSpec((1, Nt), lambda j: (0, j)),
    ]
    if has_res:
        in_specs.append(pl.BlockSpec((M, Nt), lambda j: (0, j)))
    return pl.pallas_call(
        _mb_kernel(relu, has_res),
        out_shape=jax.ShapeDtypeStruct((M, N), jnp.float32),
        grid=(N // Nt,),
        in_specs=in_specs,
        out_specs=pl.BlockSpec((M, Nt), lambda j: (0, j)),
        compiler_params=pltpu.CompilerParams(
            dimension_semantics=("parallel",),
        ),
    )


def _mb(x, w, bias, relu, residual=None):
    M, K = x.shape
    N = w.shape[1]
    fn = _get_mb(int(M), int(K), int(N), bool(relu), residual is not None)
    args = [x.astype(_BF), w.astype(_BF), bias.reshape(1, N).astype(_F32)]
    if residual is not None:
        args.append(residual.reshape(M, N).astype(_F32))
    return fn(*args)


def _conv_mb(x, w, bias, kh, kw, stride, pad, relu, residual=None):
    B, H, W, C = x.shape
    Ho = (H + 2 * pad - kh) // stride + 1
    Wo = (W + 2 * pad - kw) // stride + 1
    xp = jnp.pad(x, ((0, 0), (pad, pad), (pad, pad), (0, 0))) if pad > 0 else x
    cols = []
    for di in range(kh):
        for dj in range(kw):
            cols.append(xp[:, di:di + stride * Ho:stride,
                            dj:dj + stride * Wo:stride, :])
    patches = jnp.concatenate(cols, axis=-1) if len(cols) > 1 else cols[0]
    pm = patches.reshape(B * Ho * Wo, kh * kw * C)
    res = None
    if residual is not None:
        res = residual.reshape(B * Ho * Wo, w.shape[1])
    out = _mb(pm, w, bias, relu, res)
    return out.reshape(B, Ho, Wo, w.shape[1])


def _stem_body(p_ref, w_ref, b_ref, o_ref):
    TB = o_ref.shape[0]
    conv = _mm(p_ref[...], w_ref[...], b_ref[...], relu=True)
    conv = conv.reshape(TB, 32, 32, 64)
    xp = jnp.pad(conv, ((0, 0), (1, 1), (1, 1), (0, 0)))
    m = None
    for di in range(3):
        for dj in range(3):
            tap = xp[:, di:di + 32, dj:dj + 32, :].reshape(
                TB, 16, 2, 16, 2, 64)[:, :, 0, :, 0, :]
            m = tap if m is None else jnp.maximum(m, tap)
    o_ref[...] = m


def _stem(patches, w, b):
    TB = patches.shape[0] // 1024
    return pl.pallas_call(
        _stem_body,
        in_specs=[pl.BlockSpec(memory_space=pltpu.MemorySpace.VMEM)] * 3,
        out_specs=pl.BlockSpec(memory_space=pltpu.MemorySpace.VMEM),
        out_shape=jax.ShapeDtypeStruct((TB, 16, 16, 64), _F32),
    )(patches, w, b)


def _stem_patches(x):
    TB = x.shape[0]
    xp = jnp.pad(x, ((0, 0), (3, 3), (3, 3), (0, 0)))
    cols = []
    for di in range(7):
        for dj in range(7):
            cols.append(xp[:, di:di + 64:2, dj:dj + 64:2, :])
    return jnp.concatenate(cols, axis=-1).astype(_BF).reshape(TB * 1024, 98)


def _block0_body(stride):
    def body(x_ref, w1_ref, b1_ref, w2_ref, b2_ref, w3_ref, b3_ref,
             r_ref, o_ref):
        x4 = x_ref[...]
        B, H, W, C = x4.shape
        h1 = _mm(x4.reshape(B * H * W, C).astype(_BF), w1_ref[...],
                 b1_ref[...], relu=True)
        h1 = h1.astype(_BF).reshape(B, H, W, -1)
        h2 = _mm(_conv3x3_taps(h1, stride), w2_ref[...], b2_ref[...],
                 relu=True)
        Mo = h2.shape[0]
        h3 = _mm(h2.astype(_BF), w3_ref[...], b3_ref[...], relu=True,
                 residual=r_ref[...].reshape(Mo, -1))
        o_ref[...] = h3.reshape(o_ref.shape)
    return body


def _block0(x, w1, b1, w2, b2, w3, b3, wd, bd, stride):
    TB, H, W, C = x.shape
    P = w1.shape[-1]
    N3 = w3.shape[-1]
    Ho = (H - 1) // stride + 1
    Wo = (W - 1) // stride + 1
    if stride == 2:
        xd = x[:, ::2, ::2, :]
    else:
        xd = x
    idn = _mb(xd.reshape(TB * Ho * Wo, C), wd, bd, relu=False)
    return pl.pallas_call(
        _block0_body(stride),
        in_specs=[pl.BlockSpec(memory_space=pltpu.MemorySpace.VMEM)] * 8,
        out_specs=pl.BlockSpec(memory_space=pltpu.MemorySpace.VMEM),
        out_shape=jax.ShapeDtypeStruct((TB, Ho, Wo, N3), _F32),
    )(x, w1, b1.reshape(1, P), w2, b2.reshape(1, P), w3, b3.reshape(1, N3),
      idn.reshape(TB, Ho, Wo, N3))


def _rest_body(nb1):
    def body(x_ref, w1_ref, b1_ref, w2_ref, b2_ref, w3_ref, b3_ref,
             o_ref, xs_ref):
        b = pl.program_id(0)

        @pl.when(b == 0)
        def _():
            xs_ref[...] = x_ref[...]

        x4 = xs_ref[...]
        B, H, W, C = x4.shape
        M = B * H * W
        h1 = _mm(x4.reshape(M, C).astype(_BF), w1_ref[0], b1_ref[0],
                 relu=True)
        h1 = h1.astype(_BF).reshape(B, H, W, -1)
        h2 = _mm(_conv3x3_taps(h1, 1), w2_ref[0], b2_ref[0], relu=True)
        h3 = _mm(h2.astype(_BF), w3_ref[0], b3_ref[0], relu=True,
                 residual=x4.reshape(M, C))
        out = h3.reshape(B, H, W, C)
        xs_ref[...] = out

        @pl.when(b == nb1 - 1)
        def _():
            o_ref[...] = out
    return body


def _stage_rest(x, w1s, b1s, w2s, b2s, w3s, b3s):
    TB, H, W, C = x.shape
    nb1, _, P = w1s.shape
    return pl.pallas_call(
        _rest_body(nb1),
        grid=(nb1,),
        in_specs=[
            pl.BlockSpec((TB, H, W, C), lambda b: (0, 0, 0, 0)),
            pl.BlockSpec((1, C, P), lambda b: (b, 0, 0)),
            pl.BlockSpec((1, 1, P), lambda b: (b, 0, 0)),
            pl.BlockSpec((1, 9 * P, P), lambda b: (b, 0, 0)),
            pl.BlockSpec((1, 1, P), lambda b: (b, 0, 0)),
            pl.BlockSpec((1, P, C), lambda b: (b, 0, 0)),
            pl.BlockSpec((1, 1, C), lambda b: (b, 0, 0)),
        ],
        out_specs=pl.BlockSpec((TB, H, W, C), lambda b: (0, 0, 0, 0)),
        out_shape=jax.ShapeDtypeStruct((TB, H, W, C), _F32),
        scratch_shapes=[pltpu.VMEM((TB, H, W, C), _F32)],
        compiler_params=pltpu.CompilerParams(
            dimension_semantics=("arbitrary",)),
    )(x, w1s, b1s, w2s, b2s, w3s, b3s)


@functools.lru_cache(maxsize=None)
def _get_fc_mse(M, K, C, nb):
    inv_n = 1.0 / float(nb * C)

    def _kernel_body(x_ref, w_ref, b_ref, o_ref):
        logits = jnp.dot(x_ref[...], w_ref[...],
                         preferred_element_type=jnp.float32)
        logits = logits + b_ref[...]
        d = logits[:nb] - logits[nb:]
        o_ref[0] = jnp.sum(d * d) * inv_n

    return pl.pallas_call(
        _kernel_body,
        out_shape=jax.ShapeDtypeStruct((1,), jnp.float32),
        in_specs=[pl.BlockSpec(memory_space=pltpu.MemorySpace.VMEM),
                  pl.BlockSpec(memory_space=pltpu.MemorySpace.VMEM),
                  pl.BlockSpec(memory_space=pltpu.MemorySpace.VMEM)],
        out_specs=pl.BlockSpec(memory_space=pltpu.MemorySpace.SMEM),
    )


_LAYER_CFG = ((3, 64), (4, 128), (23, 256), (3, 512))


def _taps_conv_body(stride):
    def body(x_ref, w_ref, b_ref, o_ref):
        h1 = x_ref[...].astype(_BF)
        h2 = _mm(_conv3x3_taps(h1, stride), w_ref[...], b_ref[...],
                 relu=True)
        o_ref[...] = h2.reshape(o_ref.shape)
    return body


def _taps_conv(x, w, b, stride):
    TB, H, W, C = x.shape
    P = w.shape[-1]
    Ho = (H - 1) // stride + 1
    return pl.pallas_call(
        _taps_conv_body(stride),
        in_specs=[pl.BlockSpec(memory_space=pltpu.MemorySpace.VMEM)] * 3,
        out_specs=pl.BlockSpec(memory_space=pltpu.MemorySpace.VMEM),
        out_shape=jax.ShapeDtypeStruct((TB, Ho, Ho, P), _F32),
    )(x, w, b.reshape(1, P))


def _bottleneck_ref(x, bp, stride, taps_ok=True):
    out = _conv_mb(x, bp['w1'], bp['b1'], 1, 1, 1, 0, relu=True)
    if taps_ok:
        out = _taps_conv(out, bp['w2'], bp['b2'], stride)
    else:
        out = _conv_mb(out, bp['w2'], bp['b2'], 3, 3, stride, 1, relu=True)
    if 'wd' in bp:
        identity = _conv_mb(x, bp['wd'], bp['bd'], 1, 1, stride, 0,
                            relu=False)
    else:
        identity = x
    return _conv_mb(out, bp['w3'], bp['b3'], 1, 1, 1, 0, relu=True,
                    residual=identity)


def kernel(conv1_w, conv1_b, l0b0_w1, l0b0_b1, l0b0_w2, l0b0_b2, l0b0_w3, l0b0_b3, l0b0_wd, l0b0_bd, l0b1_w1, l0b1_b1, l0b1_w2, l0b1_b2, l0b1_w3, l0b1_b3, l0b2_w1, l0b2_b1, l0b2_w2, l0b2_b2, l0b2_w3, l0b2_b3, l1b0_w1, l1b0_b1, l1b0_w2, l1b0_b2, l1b0_w3, l1b0_b3, l1b0_wd, l1b0_bd, l1b1_w1, l1b1_b1, l1b1_w2, l1b1_b2, l1b1_w3, l1b1_b3, l1b2_w1, l1b2_b1, l1b2_w2, l1b2_b2, l1b2_w3, l1b2_b3, l1b3_w1, l1b3_b1, l1b3_w2, l1b3_b2, l1b3_w3, l1b3_b3, l2b0_w1, l2b0_b1, l2b0_w2, l2b0_b2, l2b0_w3, l2b0_b3, l2b0_wd, l2b0_bd, l2b1_w1, l2b1_b1, l2b1_w2, l2b1_b2, l2b1_w3, l2b1_b3, l2b2_w1, l2b2_b1, l2b2_w2, l2b2_b2, l2b2_w3, l2b2_b3, l2b3_w1, l2b3_b1, l2b3_w2, l2b3_b2, l2b3_w3, l2b3_b3, l2b4_w1, l2b4_b1, l2b4_w2, l2b4_b2, l2b4_w3, l2b4_b3, l2b5_w1, l2b5_b1, l2b5_w2, l2b5_b2, l2b5_w3, l2b5_b3, l2b6_w1, l2b6_b1, l2b6_w2, l2b6_b2, l2b6_w3, l2b6_b3, l2b7_w1, l2b7_b1, l2b7_w2, l2b7_b2, l2b7_w3, l2b7_b3, l2b8_w1, l2b8_b1, l2b8_w2, l2b8_b2, l2b8_w3, l2b8_b3, l2b9_w1, l2b9_b1, l2b9_w2, l2b9_b2, l2b9_w3, l2b9_b3, l2b10_w1, l2b10_b1, l2b10_w2, l2b10_b2, l2b10_w3, l2b10_b3, l2b11_w1, l2b11_b1, l2b11_w2, l2b11_b2, l2b11_w3, l2b11_b3, l2b12_w1, l2b12_b1, l2b12_w2, l2b12_b2, l2b12_w3, l2b12_b3, l2b13_w1, l2b13_b1, l2b13_w2, l2b13_b2, l2b13_w3, l2b13_b3, l2b14_w1, l2b14_b1, l2b14_w2, l2b14_b2, l2b14_w3, l2b14_b3, l2b15_w1, l2b15_b1, l2b15_w2, l2b15_b2, l2b15_w3, l2b15_b3, l2b16_w1, l2b16_b1, l2b16_w2, l2b16_b2, l2b16_w3, l2b16_b3, l2b17_w1, l2b17_b1, l2b17_w2, l2b17_b2, l2b17_w3, l2b17_b3, l2b18_w1, l2b18_b1, l2b18_w2, l2b18_b2, l2b18_w3, l2b18_b3, l2b19_w1, l2b19_b1, l2b19_w2, l2b19_b2, l2b19_w3, l2b19_b3, l2b20_w1, l2b20_b1, l2b20_w2, l2b20_b2, l2b20_w3, l2b20_b3, l2b21_w1, l2b21_b1, l2b21_w2, l2b21_b2, l2b21_w3, l2b21_b3, l2b22_w1, l2b22_b1, l2b22_w2, l2b22_b2, l2b22_w3, l2b22_b3, l3b0_w1, l3b0_b1, l3b0_w2, l3b0_b2, l3b0_w3, l3b0_b3, l3b0_wd, l3b0_bd, l3b1_w1, l3b1_b1, l3b1_w2, l3b1_b2, l3b1_w3, l3b1_b3, l3b2_w1, l3b2_b1, l3b2_w2, l3b2_b2, l3b2_w3, l3b2_b3, fc_w, fc_b, output, label):
    _L = locals()
    nb = int(output.shape[0])

    x = jnp.concatenate([output, label], axis=0).astype(_F32)
    x = jnp.transpose(x, (0, 2, 3, 1))

    x = _stem(_stem_patches(x), conv1_w, conv1_b.reshape(1, 64))

    for li, (nblocks, planes) in enumerate(_LAYER_CFG):
        stride = 1 if li == 0 else 2
        if li == 0:
            P, C = planes, planes * 4
            x = _block0(x,
                        _L["l0b0_w1"], _L["l0b0_b1"], _L["l0b0_w2"],
                        _L["l0b0_b2"], _L["l0b0_w3"], _L["l0b0_b3"],
                        _L["l0b0_wd"], _L["l0b0_bd"], stride)
            for b in range(1, nblocks):
                x = _stage_rest(x,
                                _L[f"l0b{b}_w1"].reshape(1, C, P),
                                _L[f"l0b{b}_b1"].reshape(1, 1, P),
                                _L[f"l0b{b}_w2"].reshape(1, 9 * P, P),
                                _L[f"l0b{b}_b2"].reshape(1, 1, P),
                                _L[f"l0b{b}_w3"].reshape(1, P, C),
                                _L[f"l0b{b}_b3"].reshape(1, 1, C))
            continue
        for bi in range(nblocks):
            bp = {'w1': _L[f"l{li}b{bi}_w1"], 'b1': _L[f"l{li}b{bi}_b1"],
                  'w2': _L[f"l{li}b{bi}_w2"], 'b2': _L[f"l{li}b{bi}_b2"],
                  'w3': _L[f"l{li}b{bi}_w3"], 'b3': _L[f"l{li}b{bi}_b3"]}
            if f"l{li}b{bi}_wd" in _L and bi == 0:
                bp['wd'] = _L[f"l{li}b0_wd"]
                bp['bd'] = _L[f"l{li}b0_bd"]
            taps_ok = not (li == 3 and bi == 0)
            x = _bottleneck_ref(x, bp, stride if bi == 0 else 1, taps_ok)

    feats = jnp.mean(x, axis=(1, 2))
    M, K = int(feats.shape[0]), int(feats.shape[1])
    C = int(fc_w.shape[1])
    fn = _get_fc_mse(M, K, C, nb)
    return fn(feats.astype(_BF), fc_w.astype(_BF),
              fc_b.reshape(1, C).astype(_F32))[0]
